# baseline (speedup 1.0000x reference)
import functools

import jax
import jax.numpy as jnp
from jax.experimental import pallas as pl
from jax.experimental.pallas import tpu as pltpu

BLOCK_B = 512


def _router_kernel(x_ref, w1_ref, b1_ref, w2_ref, b2_ref, prob_ref, logit_ref):
    h = jax.nn.sigmoid(
        jnp.dot(
            x_ref[...].astype(jnp.bfloat16),
            w1_ref[...].astype(jnp.bfloat16),
            preferred_element_type=jnp.float32,
        )
        + b1_ref[...]
    )
    logits = (
        jnp.dot(
            h.astype(jnp.bfloat16),
            w2_ref[...].astype(jnp.bfloat16),
            preferred_element_type=jnp.float32,
        )
        + b2_ref[...]
    )
    logit_ref[...] = logits
    m = jnp.max(logits, axis=1, keepdims=True)
    e = jnp.exp(logits - m)
    prob_ref[...] = e / jnp.sum(e, axis=1, keepdims=True)


@jax.jit
def kernel(x, W1, b1, W2, b2):
    B, D = x.shape
    H = W1.shape[1]
    E = W2.shape[1]
    b1 = b1.reshape(1, H)
    b2 = b2.reshape(1, E)
    grid = (B // BLOCK_B,)
    probs, logits = pl.pallas_call(
        _router_kernel,
        grid=grid,
        in_specs=[
            pl.BlockSpec((BLOCK_B, D), lambda i: (i, 0)),
            pl.BlockSpec((D, H), lambda i: (0, 0)),
            pl.BlockSpec((1, H), lambda i: (0, 0)),
            pl.BlockSpec((H, E), lambda i: (0, 0)),
            pl.BlockSpec((1, E), lambda i: (0, 0)),
        ],
        out_specs=[
            pl.BlockSpec((BLOCK_B, E), lambda i: (i, 0)),
            pl.BlockSpec((BLOCK_B, E), lambda i: (i, 0)),
        ],
        out_shape=[
            jax.ShapeDtypeStruct((B, E), jnp.float32),
            jax.ShapeDtypeStruct((B, E), jnp.float32),
        ],
        compiler_params=pltpu.CompilerParams(
            dimension_semantics=("parallel",),
        ),
    )(x, W1, b1, W2, b2)
    return (probs, logits)


# speedup vs baseline: 1.1681x; 1.1681x over previous
import functools

import jax
import jax.numpy as jnp
from jax.experimental import pallas as pl
from jax.experimental.pallas import tpu as pltpu

BLOCK_B = 1024


def _router_kernel(x_ref, w1_ref, b1_ref, w2_ref, b2_ref, prob_ref, logit_ref):
    h = jax.nn.sigmoid(
        jnp.dot(
            x_ref[...].astype(jnp.bfloat16),
            w1_ref[...].astype(jnp.bfloat16),
            preferred_element_type=jnp.float32,
        )
        + b1_ref[...]
    )
    logits = (
        jnp.dot(
            h.astype(jnp.bfloat16),
            w2_ref[...].astype(jnp.bfloat16),
            preferred_element_type=jnp.float32,
        )
        + b2_ref[...]
    )
    logit_ref[...] = logits
    m = jnp.max(logits, axis=1, keepdims=True)
    e = jnp.exp(logits - m)
    prob_ref[...] = e / jnp.sum(e, axis=1, keepdims=True)


@jax.jit
def kernel(x, W1, b1, W2, b2):
    B, D = x.shape
    H = W1.shape[1]
    E = W2.shape[1]
    b1 = b1.reshape(1, H)
    b2 = b2.reshape(1, E)
    grid = (B // BLOCK_B,)
    probs, logits = pl.pallas_call(
        _router_kernel,
        grid=grid,
        in_specs=[
            pl.BlockSpec((BLOCK_B, D), lambda i: (i, 0)),
            pl.BlockSpec((D, H), lambda i: (0, 0)),
            pl.BlockSpec((1, H), lambda i: (0, 0)),
            pl.BlockSpec((H, E), lambda i: (0, 0)),
            pl.BlockSpec((1, E), lambda i: (0, 0)),
        ],
        out_specs=[
            pl.BlockSpec((BLOCK_B, E), lambda i: (i, 0)),
            pl.BlockSpec((BLOCK_B, E), lambda i: (i, 0)),
        ],
        out_shape=[
            jax.ShapeDtypeStruct((B, E), jnp.float32),
            jax.ShapeDtypeStruct((B, E), jnp.float32),
        ],
        compiler_params=pltpu.CompilerParams(
            dimension_semantics=("parallel",),
        ),
    )(x, W1, b1, W2, b2)
    return (probs, logits)
